# baseline (speedup 1.0000x reference)
import functools

import jax
import jax.numpy as jnp
from jax import lax
from jax.experimental import pallas as pl
from jax.experimental.pallas import tpu as pltpu


def _attn_body(q_ref, k_ref, v_ref, nk_ref, nv_ref, w_ref, o_ref, *,
               h, hh, t, d, buckets, bsz, scale):
    pid = pl.program_id(0)
    rolled = (pid % h) >= hh

    qq = q_ref[0] * scale
    kk = k_ref[0]
    vv = v_ref[0]
    shift = bsz - 1
    q2, k2, v2 = lax.cond(
        rolled,
        lambda: (jnp.roll(qq, -shift, axis=0),
                 jnp.roll(kk, -shift, axis=0),
                 jnp.roll(vv, -shift, axis=0)),
        lambda: (qq, kk, vv))

    b_q = q2.reshape(buckets, bsz, d)
    b_k = k2.reshape(buckets, bsz, d)
    b_v = v2.reshape(buckets, bsz, d)

    bucket_sums = jnp.sum(b_k, axis=1)
    ri32 = lax.broadcasted_iota(jnp.int32, (buckets, buckets), 0)
    cj32 = lax.broadcasted_iota(jnp.int32, (buckets, buckets), 1)
    tril = (cj32 < ri32).astype(jnp.float32)
    prefix_excl = jnp.dot(tril, bucket_sums, preferred_element_type=jnp.float32)
    first = b_k[:, 0]
    csum = prefix_excl + first
    denom = (lax.broadcasted_iota(jnp.int32, (buckets, 1), 0) * bsz + 1
             ).astype(jnp.float32)
    x = jnp.concatenate([csum / denom, first], axis=-1)

    r = jnp.dot(x, w_ref[0], preferred_element_type=jnp.float32)
    r = jnp.where(r >= 0, r, 0.01 * r)
    ri = lax.broadcasted_iota(jnp.int32, (buckets, buckets + 1), 0)
    cj = lax.broadcasted_iota(jnp.int32, (buckets, buckets + 1), 1)
    neg = -jnp.finfo(jnp.float32).max
    r = jnp.where(cj > ri, neg, r)
    r = jax.nn.softmax(r, axis=-1)
    r = jnp.where(cj < ri, r, 0.0)
    w_top = jnp.max(r, axis=-1, keepdims=True)
    idx = jnp.argmax(r, axis=-1)
    r_top = jnp.where(cj == idx[:, None], w_top, 0.0)

    null_k = jnp.broadcast_to(nk_ref[0, 0][None, None, :], (1, bsz, d))
    null_v = jnp.broadcast_to(nv_ref[0, 0][None, None, :], (1, bsz, d))
    cat_k = jnp.concatenate([null_k, b_k], axis=0).reshape(buckets + 1, bsz * d)
    cat_v = jnp.concatenate([null_v, b_v], axis=0).reshape(buckets + 1, bsz * d)
    b_k_r = jnp.dot(r_top, cat_k, preferred_element_type=jnp.float32)
    b_v_r = jnp.dot(r_top, cat_v, preferred_element_type=jnp.float32)
    b_k_r = b_k_r.reshape(buckets, bsz, d)
    b_v_r = b_v_r.reshape(buckets, bsz, d)

    dots_r = lax.dot_general(b_q, b_k_r, (((2,), (2,)), ((0,), (0,))),
                             preferred_element_type=jnp.float32)
    dots_o = lax.dot_general(b_q, b_k, (((2,), (2,)), ((0,), (0,))),
                             preferred_element_type=jnp.float32)
    dots = jnp.concatenate([dots_r, dots_o], axis=-1)

    row = lax.broadcasted_iota(jnp.int32, (bsz, 2 * bsz), 0)
    col = lax.broadcasted_iota(jnp.int32, (bsz, 2 * bsz), 1)
    base = jnp.logical_not((col >= bsz) & ((col - bsz) > row))
    ub = lax.broadcasted_iota(jnp.int32, (buckets, 1, 1), 0)
    use_sp = jnp.logical_and(rolled, ub == buckets - 1)
    restrict = jnp.logical_and(use_sp, (col <= bsz)[None])
    mask = base[None] & (jnp.logical_not(restrict) | (row == 0)[None])
    dots = jnp.where(mask, dots, neg)
    dots = jax.nn.softmax(dots, axis=-1)
    p_r = dots[:, :, :bsz]
    p_o = dots[:, :, bsz:]
    out = (lax.dot_general(p_r, b_v_r, (((2,), (1,)), ((0,), (0,))),
                           preferred_element_type=jnp.float32)
           + lax.dot_general(p_o, b_v, (((2,), (1,)), ((0,), (0,))),
                             preferred_element_type=jnp.float32))

    o = out.reshape(t, d)
    o = lax.cond(rolled, lambda: jnp.roll(o, shift, axis=0), lambda: o)
    o_ref[0] = o


def kernel(q, k, v, null_keys, null_values, sort_linear):
    b, h, t, d = q.shape
    buckets = sort_linear.shape[-1] - 1
    bsz = t // buckets
    hh = h // 2
    bh = b * h
    scale = float((h * d) ** -0.5)

    qr = q.reshape(bh, t, d)
    kr = k.reshape(bh, t, d)
    vr = v.reshape(bh, t, d)
    nk = null_keys.reshape(h, 1, d)
    nv = null_values.reshape(h, 1, d)
    W = sort_linear.reshape(h, 2 * d, buckets + 1)

    body = functools.partial(_attn_body, h=h, hh=hh, t=t, d=d,
                             buckets=buckets, bsz=bsz, scale=scale)

    out = pl.pallas_call(
        body,
        grid=(bh,),
        in_specs=[
            pl.BlockSpec((1, t, d), lambda i: (i, 0, 0)),
            pl.BlockSpec((1, t, d), lambda i: (i, 0, 0)),
            pl.BlockSpec((1, t, d), lambda i: (i, 0, 0)),
            pl.BlockSpec((1, 1, d), lambda i: (i % h, 0, 0)),
            pl.BlockSpec((1, 1, d), lambda i: (i % h, 0, 0)),
            pl.BlockSpec((1, 2 * d, buckets + 1), lambda i: (i % h, 0, 0)),
        ],
        out_specs=pl.BlockSpec((1, t, d), lambda i: (i, 0, 0)),
        out_shape=jax.ShapeDtypeStruct((bh, t, d), jnp.float32),
        compiler_params=pltpu.CompilerParams(
            dimension_semantics=("parallel",)),
    )(qr, kr, vr, nk, nv, W)
    return out.reshape(b, h, t, d)


# speedup vs baseline: 1.0786x; 1.0786x over previous
import functools

import jax
import jax.numpy as jnp
from jax import lax
from jax.experimental import pallas as pl
from jax.experimental.pallas import tpu as pltpu


def _attn_body(q_ref, k_ref, v_ref, nk_ref, nv_ref, w_ref, o_ref, *,
               h, hh, t, d, buckets, bsz, scale):
    pid = pl.program_id(0)
    rolled = (pid % h) >= hh

    qq = q_ref[0] * scale
    kk = k_ref[0]
    vv = v_ref[0]
    shift = bsz - 1
    q2 = jnp.where(rolled, jnp.roll(qq, -shift, axis=0), qq)
    k2 = jnp.where(rolled, jnp.roll(kk, -shift, axis=0), kk)
    v2 = jnp.where(rolled, jnp.roll(vv, -shift, axis=0), vv)

    b_q = q2.reshape(buckets, bsz, d)
    b_k = k2.reshape(buckets, bsz, d)
    b_v = v2.reshape(buckets, bsz, d)

    bucket_sums = jnp.sum(b_k, axis=1)
    ri32 = lax.broadcasted_iota(jnp.int32, (buckets, buckets), 0)
    cj32 = lax.broadcasted_iota(jnp.int32, (buckets, buckets), 1)
    tril = (cj32 < ri32).astype(jnp.float32)
    prefix_excl = jnp.dot(tril, bucket_sums, preferred_element_type=jnp.float32)
    first = b_k[:, 0]
    csum = prefix_excl + first
    denom = (lax.broadcasted_iota(jnp.int32, (buckets, 1), 0) * bsz + 1
             ).astype(jnp.float32)
    x = jnp.concatenate([csum / denom, first], axis=-1)

    r = jnp.dot(x, w_ref[0], preferred_element_type=jnp.float32)
    r = jnp.where(r >= 0, r, 0.01 * r)
    ri = lax.broadcasted_iota(jnp.int32, (buckets, buckets + 1), 0)
    cj = lax.broadcasted_iota(jnp.int32, (buckets, buckets + 1), 1)
    neg = -jnp.finfo(jnp.float32).max
    r = jnp.where(cj > ri, neg, r)
    r = jax.nn.softmax(r, axis=-1)
    r = jnp.where(cj < ri, r, 0.0)
    w_top = jnp.max(r, axis=-1, keepdims=True)
    idx = jnp.argmax(r, axis=-1)
    r_top = jnp.where(cj == idx[:, None], w_top, 0.0)

    null_k = jnp.broadcast_to(nk_ref[0, 0][None, None, :], (1, bsz, d))
    null_v = jnp.broadcast_to(nv_ref[0, 0][None, None, :], (1, bsz, d))
    cat_k = jnp.concatenate([null_k, b_k], axis=0).reshape(buckets + 1, bsz * d)
    cat_v = jnp.concatenate([null_v, b_v], axis=0).reshape(buckets + 1, bsz * d)
    b_k_r = jnp.dot(r_top, cat_k, preferred_element_type=jnp.float32)
    b_v_r = jnp.dot(r_top, cat_v, preferred_element_type=jnp.float32)
    b_k_r = b_k_r.reshape(buckets, bsz, d)
    b_v_r = b_v_r.reshape(buckets, bsz, d)

    dots_r = lax.dot_general(b_q, b_k_r, (((2,), (2,)), ((0,), (0,))),
                             preferred_element_type=jnp.float32)
    dots_o = lax.dot_general(b_q, b_k, (((2,), (2,)), ((0,), (0,))),
                             preferred_element_type=jnp.float32)
    dots = jnp.concatenate([dots_r, dots_o], axis=-1)

    row = lax.broadcasted_iota(jnp.int32, (bsz, 2 * bsz), 0)
    col = lax.broadcasted_iota(jnp.int32, (bsz, 2 * bsz), 1)
    base = jnp.logical_not((col >= bsz) & ((col - bsz) > row))
    ub = lax.broadcasted_iota(jnp.int32, (buckets, 1, 1), 0)
    use_sp = jnp.logical_and(rolled, ub == buckets - 1)
    restrict = jnp.logical_and(use_sp, (col <= bsz)[None])
    mask = base[None] & (jnp.logical_not(restrict) | (row == 0)[None])
    dots = jnp.where(mask, dots, neg)
    dots = jax.nn.softmax(dots, axis=-1)
    p_r = dots[:, :, :bsz]
    p_o = dots[:, :, bsz:]
    out = (lax.dot_general(p_r, b_v_r, (((2,), (1,)), ((0,), (0,))),
                           preferred_element_type=jnp.float32)
           + lax.dot_general(p_o, b_v, (((2,), (1,)), ((0,), (0,))),
                             preferred_element_type=jnp.float32))

    o = out.reshape(t, d)
    o = jnp.where(rolled, jnp.roll(o, shift, axis=0), o)
    o_ref[0] = o


def kernel(q, k, v, null_keys, null_values, sort_linear):
    b, h, t, d = q.shape
    buckets = sort_linear.shape[-1] - 1
    bsz = t // buckets
    hh = h // 2
    bh = b * h
    scale = float((h * d) ** -0.5)

    qr = q.reshape(bh, t, d)
    kr = k.reshape(bh, t, d)
    vr = v.reshape(bh, t, d)
    nk = null_keys.reshape(h, 1, d)
    nv = null_values.reshape(h, 1, d)
    W = sort_linear.reshape(h, 2 * d, buckets + 1)

    body = functools.partial(_attn_body, h=h, hh=hh, t=t, d=d,
                             buckets=buckets, bsz=bsz, scale=scale)

    out = pl.pallas_call(
        body,
        grid=(bh,),
        in_specs=[
            pl.BlockSpec((1, t, d), lambda i: (i, 0, 0)),
            pl.BlockSpec((1, t, d), lambda i: (i, 0, 0)),
            pl.BlockSpec((1, t, d), lambda i: (i, 0, 0)),
            pl.BlockSpec((1, 1, d), lambda i: (i % h, 0, 0)),
            pl.BlockSpec((1, 1, d), lambda i: (i % h, 0, 0)),
            pl.BlockSpec((1, 2 * d, buckets + 1), lambda i: (i % h, 0, 0)),
        ],
        out_specs=pl.BlockSpec((1, t, d), lambda i: (i, 0, 0)),
        out_shape=jax.ShapeDtypeStruct((bh, t, d), jnp.float32),
        compiler_params=pltpu.CompilerParams(
            dimension_semantics=("parallel",)),
    )(qr, kr, vr, nk, nv, W)
    return out.reshape(b, h, t, d)


# speedup vs baseline: 1.8275x; 1.6943x over previous
import functools

import jax
import jax.numpy as jnp
from jax import lax
from jax.experimental import pallas as pl
from jax.experimental.pallas import tpu as pltpu


def _attn_body(q_ref, k_ref, v_ref, nk_ref, nv_ref, w_ref, o_ref, *,
               h, hh, t, d, buckets, bsz, scale):
    pid = pl.program_id(0)
    rolled = (pid % h) >= hh

    qq = q_ref[0] * scale
    kk = k_ref[0]
    vv = v_ref[0]
    shift = bsz - 1
    q2 = jnp.where(rolled, jnp.roll(qq, -shift, axis=0), qq)
    k2 = jnp.where(rolled, jnp.roll(kk, -shift, axis=0), kk)
    v2 = jnp.where(rolled, jnp.roll(vv, -shift, axis=0), vv)

    b_q = q2.reshape(buckets, bsz, d)
    b_k = k2.reshape(buckets, bsz, d)
    b_v = v2.reshape(buckets, bsz, d)

    bucket_sums = jnp.sum(b_k, axis=1)
    ri32 = lax.broadcasted_iota(jnp.int32, (buckets, buckets), 0)
    cj32 = lax.broadcasted_iota(jnp.int32, (buckets, buckets), 1)
    tril = (cj32 < ri32).astype(jnp.float32)
    prefix_excl = jnp.dot(tril, bucket_sums, preferred_element_type=jnp.float32)
    first = b_k[:, 0]
    csum = prefix_excl + first
    denom = (lax.broadcasted_iota(jnp.int32, (buckets, 1), 0) * bsz + 1
             ).astype(jnp.float32)
    x = jnp.concatenate([csum / denom, first], axis=-1)

    r = jnp.dot(x, w_ref[0], preferred_element_type=jnp.float32)
    r = jnp.where(r >= 0, r, 0.01 * r)
    ri = lax.broadcasted_iota(jnp.int32, (buckets, buckets + 1), 0)
    cj = lax.broadcasted_iota(jnp.int32, (buckets, buckets + 1), 1)
    neg = -jnp.finfo(jnp.float32).max
    r = jnp.where(cj > ri, neg, r)
    r = jax.nn.softmax(r, axis=-1)
    r = jnp.where(cj < ri, r, 0.0)
    w_top = jnp.max(r, axis=-1, keepdims=True)
    idx = jnp.argmax(r, axis=-1)
    r_top = jnp.where(cj == idx[:, None], w_top, 0.0)

    null_k = jnp.broadcast_to(nk_ref[0, 0][None, None, :], (1, bsz, d))
    null_v = jnp.broadcast_to(nv_ref[0, 0][None, None, :], (1, bsz, d))
    cat_k = jnp.concatenate([null_k, b_k], axis=0).reshape(buckets + 1, bsz * d)
    cat_v = jnp.concatenate([null_v, b_v], axis=0).reshape(buckets + 1, bsz * d)
    b_k_r = jnp.dot(r_top, cat_k, preferred_element_type=jnp.float32)
    b_v_r = jnp.dot(r_top, cat_v, preferred_element_type=jnp.float32)
    b_k_r = b_k_r.reshape(buckets, bsz, d)
    b_v_r = b_v_r.reshape(buckets, bsz, d)

    b_k2 = jnp.concatenate([b_k_r, b_k], axis=1)
    b_v2 = jnp.concatenate([b_v_r, b_v], axis=1)
    dots = lax.dot_general(b_q, b_k2, (((2,), (2,)), ((0,), (0,))),
                           preferred_element_type=jnp.float32)

    row = lax.broadcasted_iota(jnp.int32, (bsz, 2 * bsz), 0)
    col = lax.broadcasted_iota(jnp.int32, (bsz, 2 * bsz), 1)
    base = jnp.logical_not((col >= bsz) & ((col - bsz) > row))
    ub = lax.broadcasted_iota(jnp.int32, (buckets, 1, 1), 0)
    use_sp = jnp.logical_and(rolled, ub == buckets - 1)
    restrict = jnp.logical_and(use_sp, (col <= bsz)[None])
    mask = base[None] & (jnp.logical_not(restrict) | (row == 0)[None])
    dots = jnp.where(mask, dots, neg)
    dots = jax.nn.softmax(dots, axis=-1)
    out = lax.dot_general(dots, b_v2, (((2,), (1,)), ((0,), (0,))),
                          preferred_element_type=jnp.float32)

    o = out.reshape(t, d)
    o = jnp.where(rolled, jnp.roll(o, shift, axis=0), o)
    o_ref[0] = o


def kernel(q, k, v, null_keys, null_values, sort_linear):
    b, h, t, d = q.shape
    buckets = sort_linear.shape[-1] - 1
    bsz = t // buckets
    hh = h // 2
    bh = b * h
    scale = float((h * d) ** -0.5)

    qr = q.reshape(bh, t, d)
    kr = k.reshape(bh, t, d)
    vr = v.reshape(bh, t, d)
    nk = null_keys.reshape(h, 1, d)
    nv = null_values.reshape(h, 1, d)
    W = sort_linear.reshape(h, 2 * d, buckets + 1)

    body = functools.partial(_attn_body, h=h, hh=hh, t=t, d=d,
                             buckets=buckets, bsz=bsz, scale=scale)

    out = pl.pallas_call(
        body,
        grid=(bh,),
        in_specs=[
            pl.BlockSpec((1, t, d), lambda i: (i, 0, 0)),
            pl.BlockSpec((1, t, d), lambda i: (i, 0, 0)),
            pl.BlockSpec((1, t, d), lambda i: (i, 0, 0)),
            pl.BlockSpec((1, 1, d), lambda i: (i % h, 0, 0)),
            pl.BlockSpec((1, 1, d), lambda i: (i % h, 0, 0)),
            pl.BlockSpec((1, 2 * d, buckets + 1), lambda i: (i % h, 0, 0)),
        ],
        out_specs=pl.BlockSpec((1, t, d), lambda i: (i, 0, 0)),
        out_shape=jax.ShapeDtypeStruct((bh, t, d), jnp.float32),
        compiler_params=pltpu.CompilerParams(
            dimension_semantics=("parallel",)),
    )(qr, kr, vr, nk, nv, W)
    return out.reshape(b, h, t, d)
